# baseline (speedup 1.0000x reference)
import jax
import jax.numpy as jnp
from jax.experimental import pallas as pl
from jax.experimental.pallas import tpu as pltpu

_USER = 8192
_ITEM = 8192
_DIM = 64
_TM = 256
_LAYER = 2


def _body(xt_ref, gammat_ref, betat_ref, adj_ref,
          et_ref,
          bnt_ref, bni_ref, iacct_ref, xtc_ref):
    l = pl.program_id(0)
    i = pl.program_id(1)
    ni = pl.num_programs(1)

    @pl.when((l == 0) & (i == 0))
    def _seed():
        xtc_ref[...] = xt_ref[...]

    @pl.when(i == 0)
    def _init():
        xt = xtc_ref[...]
        mean = jnp.mean(xt, axis=1, keepdims=True)
        var = jnp.mean((xt - mean) ** 2, axis=1, keepdims=True)
        s = gammat_ref[0] * jax.lax.rsqrt(var + 1e-5)
        t = betat_ref[0] - mean * s
        bnt = (xt * s + t).astype(jnp.bfloat16)
        bnt_ref[...] = bnt
        bni_ref[...] = jnp.transpose(bnt[:, _USER:])
        iacct_ref[...] = jnp.zeros_like(iacct_ref)

    a = adj_ref[...].astype(jnp.bfloat16)

    ug = jax.lax.dot_general(
        a, bni_ref[...],
        dimension_numbers=(((1,), (0,)), ((), ())),
        preferred_element_type=jnp.float32)
    ugt = jnp.transpose(ug)
    et_ref[0, :, pl.ds(i * _TM, _TM)] = ugt

    @pl.when(l == 0)
    def _carry_user():
        xtc_ref[:, pl.ds(i * _TM, _TM)] += ugt

    iacct_ref[...] += jax.lax.dot_general(
        bnt_ref[:, pl.ds(i * _TM, _TM)], a,
        dimension_numbers=(((1,), (0,)), ((), ())),
        preferred_element_type=jnp.float32)

    @pl.when(i == ni - 1)
    def _fin():
        et_ref[0, :, _USER:] = iacct_ref[...]

    @pl.when((i == ni - 1) & (l == 0))
    def _carry_item():
        xtc_ref[:, _USER:] += iacct_ref[...]


def _both_layers(adj, xt, gammat3, betat3):
    n_blk = _USER // _TM
    return pl.pallas_call(
        _body,
        grid=(_LAYER, n_blk),
        in_specs=[
            pl.BlockSpec((_DIM, _USER + _ITEM), lambda l, i: (0, 0)),
            pl.BlockSpec((1, _DIM, 1), lambda l, i: (l, 0, 0)),
            pl.BlockSpec((1, _DIM, 1), lambda l, i: (l, 0, 0)),
            pl.BlockSpec((_TM, _ITEM), lambda l, i: (i, 0)),
        ],
        out_specs=pl.BlockSpec(
            (1, _DIM, _USER + _ITEM), lambda l, i: (l, 0, 0)),
        out_shape=jax.ShapeDtypeStruct(
            (_LAYER, _DIM, _USER + _ITEM), jnp.float32),
        scratch_shapes=[
            pltpu.VMEM((_DIM, _USER + _ITEM), jnp.bfloat16),
            pltpu.VMEM((_ITEM, _DIM), jnp.bfloat16),
            pltpu.VMEM((_DIM, _ITEM), jnp.float32),
            pltpu.VMEM((_DIM, _USER + _ITEM), jnp.float32),
        ],
        compiler_params=pltpu.CompilerParams(
            dimension_semantics=("arbitrary", "arbitrary")),
    )(xt, gammat3, betat3, adj)


def kernel(adj, embeds, bn_gamma, bn_beta):
    xt = jnp.transpose(embeds)
    et = _both_layers(adj, xt, bn_gamma[:, :, None], bn_beta[:, :, None])
    e1 = jnp.transpose(et[0])
    e2 = jnp.transpose(et[1])
    x1 = embeds + e1
    x2 = x1 + e2
    return (jnp.stack([embeds, x1, x2]), jnp.stack([embeds, e1, e2]))


# speedup vs baseline: 1.0858x; 1.0858x over previous
import jax
import jax.numpy as jnp
from jax.experimental import pallas as pl
from jax.experimental.pallas import tpu as pltpu

_USER = 8192
_ITEM = 8192
_DIM = 64
_TM = 512


def _layer_body(xt_ref, gammat_ref, betat_ref, adj_ref,
                et_ref,
                bnt_ref, bni_ref, iacct_ref):
    i = pl.program_id(0)
    ni = pl.num_programs(0)

    @pl.when(i == 0)
    def _init():
        xt = xt_ref[...]
        mean = jnp.mean(xt, axis=1, keepdims=True)
        var = jnp.mean((xt - mean) ** 2, axis=1, keepdims=True)
        s = gammat_ref[...] * jax.lax.rsqrt(var + 1e-5)
        t = betat_ref[...] - mean * s
        bnt = (xt * s + t).astype(jnp.bfloat16)
        bnt_ref[...] = bnt
        bni_ref[...] = jnp.transpose(bnt[:, _USER:])
        iacct_ref[...] = jnp.zeros_like(iacct_ref)

    a = adj_ref[...].astype(jnp.bfloat16)

    ug = jax.lax.dot_general(
        a, bni_ref[...],
        dimension_numbers=(((1,), (0,)), ((), ())),
        preferred_element_type=jnp.float32)
    et_ref[:, pl.ds(i * _TM, _TM)] = jnp.transpose(ug)

    iacct_ref[...] += jax.lax.dot_general(
        bnt_ref[:, pl.ds(i * _TM, _TM)], a,
        dimension_numbers=(((1,), (0,)), ((), ())),
        preferred_element_type=jnp.float32)

    @pl.when(i == ni - 1)
    def _fin():
        et_ref[:, _USER:] = iacct_ref[...]


def _fused_layer(adj, xt, gammat, betat):
    n_blk = _USER // _TM
    return pl.pallas_call(
        _layer_body,
        grid=(n_blk,),
        in_specs=[
            pl.BlockSpec((_DIM, _USER + _ITEM), lambda i: (0, 0)),
            pl.BlockSpec((_DIM, 1), lambda i: (0, 0)),
            pl.BlockSpec((_DIM, 1), lambda i: (0, 0)),
            pl.BlockSpec((_TM, _ITEM), lambda i: (i, 0)),
        ],
        out_specs=pl.BlockSpec((_DIM, _USER + _ITEM), lambda i: (0, 0)),
        out_shape=jax.ShapeDtypeStruct((_DIM, _USER + _ITEM), jnp.float32),
        scratch_shapes=[
            pltpu.VMEM((_DIM, _USER + _ITEM), jnp.bfloat16),
            pltpu.VMEM((_ITEM, _DIM), jnp.bfloat16),
            pltpu.VMEM((_DIM, _ITEM), jnp.float32),
        ],
        compiler_params=pltpu.CompilerParams(
            dimension_semantics=("arbitrary",)),
    )(xt, gammat, betat, adj)


def kernel(adj, embeds, bn_gamma, bn_beta):
    x = embeds
    xt = jnp.transpose(embeds)
    lats = [embeds]
    gcn_lats = [embeds]
    for layer in range(2):
        gt = bn_gamma[layer][:, None]
        bt = bn_beta[layer][:, None]
        et = _fused_layer(adj, xt, gt, bt)
        e = jnp.transpose(et)
        gcn_lats.append(e)
        x = x + e
        xt = xt + et
        lats.append(x)
    return (jnp.stack(lats), jnp.stack(gcn_lats))
